# Initial kernel scaffold; baseline (speedup 1.0000x reference)
import jax
import jax.numpy as jnp
from jax.experimental import pallas as pl


def kernel(x, edge_index, edge_attr, W_l, b_l, W_r, b_r, att, W_e, b_e, bias, gate_W, gate_b):
    raise NotImplementedError("write your pallas kernel here")


# speedup vs baseline: 7.5231x; 7.5231x over previous
import functools

import jax
import jax.numpy as jnp
from jax import lax
from jax.experimental import pallas as pl
from jax.experimental.pallas import tpu as pltpu
from jax.experimental.pallas import tpu_sc as plsc

N = 10000
E = 320000
IN = 128
OUT = 128
H = 4
ED = 16
NEG = 0.2

NTILES = 32
EPT = E // NTILES
C = 80
NCHUNK = EPT // C
ROWS_PT = 624
TAIL = N - 16 * ROWS_PT
ZROWS = 48


def _hsum16(v):
    lanes = lax.iota(jnp.int32, 16)
    for off in (8, 4, 2, 1):
        idx = jnp.bitwise_xor(lanes, off)
        shuf = lax.gather(
            v, idx[:, None],
            lax.GatherDimensionNumbers(
                offset_dims=(), collapsed_slice_dims=(0,), start_index_map=(0,)),
            (1,), mode=lax.GatherScatterMode.PROMISE_IN_BOUNDS)
        v = v + shuf
    return v


def _proj_kernel(x_ref, wl_ref, bl_ref, wr_ref, br_ref, xlh_ref, xrh_ref):
    x = x_ref[...]
    xlh_ref[...] = jnp.dot(x, wl_ref[...], preferred_element_type=jnp.float32) + bl_ref[0]
    xrh_ref[...] = jnp.dot(x, wr_ref[...], preferred_element_type=jnp.float32) + br_ref[0]


def _proj(x, W_l, b_l, W_r, b_r):
    BN = 400
    grid = (H, N // BN)
    return pl.pallas_call(
        _proj_kernel,
        grid=grid,
        in_specs=[
            pl.BlockSpec((BN, IN), lambda h, i: (i, 0)),
            pl.BlockSpec((IN, OUT), lambda h, i: (0, h)),
            pl.BlockSpec((1, 1, OUT), lambda h, i: (h, 0, 0)),
            pl.BlockSpec((IN, OUT), lambda h, i: (0, h)),
            pl.BlockSpec((1, 1, OUT), lambda h, i: (h, 0, 0)),
        ],
        out_specs=[
            pl.BlockSpec((BN, OUT), lambda h, i: (h * (N // BN) + i, 0)),
            pl.BlockSpec((BN, OUT), lambda h, i: (h * (N // BN) + i, 0)),
        ],
        out_shape=[
            jax.ShapeDtypeStruct((H * N, OUT), jnp.float32),
            jax.ShapeDtypeStruct((H * N, OUT), jnp.float32),
        ],
    )(x, W_l, b_l.reshape(H, 1, OUT), W_r, b_r.reshape(H, 1, OUT))


def _eatt_kernel(ea_ref, we_ref, be_ref, out_ref):
    out_ref[...] = (
        jnp.dot(ea_ref[...], we_ref[...], preferred_element_type=jnp.float32)
        + be_ref[0]
    )


def _eatt(edge_attr, W_e, b_e):
    BE = 2000
    grid = (H, E // BE)
    return pl.pallas_call(
        _eatt_kernel,
        grid=grid,
        in_specs=[
            pl.BlockSpec((BE, ED), lambda h, i: (i, 0)),
            pl.BlockSpec((ED, OUT), lambda h, i: (0, h)),
            pl.BlockSpec((1, 1, OUT), lambda h, i: (h, 0, 0)),
        ],
        out_specs=pl.BlockSpec((BE, OUT), lambda h, i: (h * (E // BE) + i, 0)),
        out_shape=jax.ShapeDtypeStruct((H * E, OUT), jnp.float32),
    )(edge_attr, W_e, b_e.reshape(H, 1, OUT))


def _sc_head(h, xlh, xrh, eatt, src, dst, att_h):
    mesh = plsc.VectorSubcoreMesh(core_axis_name="c", subcore_axis_name="s")

    @functools.partial(
        pl.kernel,
        mesh=mesh,
        out_type=[
            jax.ShapeDtypeStruct((2 * N, OUT), jnp.float32),
            jax.ShapeDtypeStruct((E, 16), jnp.float32),
        ],
        scratch_types=[
            pltpu.VMEM((C,), jnp.int32),
            pltpu.VMEM((C,), jnp.int32),
            pltpu.VMEM((C,), jnp.int32),
            pltpu.VMEM((C,), jnp.int32),
            pltpu.VMEM((C, OUT), jnp.float32),
            pltpu.VMEM((C, OUT), jnp.float32),
            pltpu.VMEM((C, OUT), jnp.float32),
            pltpu.VMEM((C, 16), jnp.float32),
            pltpu.VMEM((OUT,), jnp.float32),
            pltpu.VMEM((ZROWS, OUT), jnp.float32),
            pltpu.VMEM_SHARED((N, OUT), jnp.float32),
            pltpu.SemaphoreType.DMA,
            pltpu.SemaphoreType.DMA,
            pltpu.SemaphoreType.DMA,
        ],
    )
    def k(xlh_hbm, xrh_hbm, ea_hbm, src_hbm, dst_hbm, att_hbm,
          accp_hbm, ex_hbm,
          srcb, dstb, idxj, idxi, xj, xi, ea, exrow, attv, zb,
          acc, sem1, sem2, sem3):
        cid = lax.axis_index("c")
        sid = lax.axis_index("s")
        wid = cid * 16 + sid
        ebase = wid * EPT

        pltpu.sync_copy(att_hbm, attv)

        def zrow(j, _):
            for g in range(OUT // 16):
                zb[j, pl.ds(g * 16, 16)] = jnp.zeros((16,), jnp.float32)
            return _

        lax.fori_loop(0, ZROWS, zrow, None)

        for t in range(ROWS_PT // ZROWS):
            pltpu.sync_copy(zb, acc.at[pl.ds(sid * ROWS_PT + t * ZROWS, ZROWS), :])

        @pl.when(sid == 15)
        def _zero_tail():
            pltpu.sync_copy(zb.at[pl.ds(0, TAIL), :],
                            acc.at[pl.ds(16 * ROWS_PT, TAIL), :])

        plsc.subcore_barrier()

        def chunk_body(i, _):
            base = ebase + i * C
            pltpu.sync_copy(src_hbm.at[pl.ds(base, C)], srcb)
            pltpu.sync_copy(dst_hbm.at[pl.ds(base, C)], dstb)

            def idx_body(g, _):
                idxj[pl.ds(g * 16, 16)] = srcb[pl.ds(g * 16, 16)] + h * N
                idxi[pl.ds(g * 16, 16)] = dstb[pl.ds(g * 16, 16)] + h * N
                return _

            lax.fori_loop(0, C // 16, idx_body, None, unroll=True)

            cj = pltpu.async_copy(xlh_hbm.at[idxj], xj, sem1)
            ci = pltpu.async_copy(xrh_hbm.at[idxi], xi, sem2)
            ce = pltpu.async_copy(ea_hbm.at[pl.ds(h * E + base, C), :], ea, sem3)
            cj.wait()
            ci.wait()
            ce.wait()

            def edge_body(j, _):
                hacc = jnp.zeros((16,), jnp.float32)
                for g in range(OUT // 16):
                    v = (xj[j, pl.ds(g * 16, 16)]
                         + xi[j, pl.ds(g * 16, 16)]
                         + ea[j, pl.ds(g * 16, 16)])
                    lr = 0.6 * v + 0.4 * jnp.abs(v)
                    hacc = hacc + lr * attv[pl.ds(g * 16, 16)]
                w = jnp.exp(_hsum16(hacc))
                exrow[j, pl.ds(0, 16)] = w
                wq = w * (1.0 / H)
                for g in range(OUT // 16):
                    xj[j, pl.ds(g * 16, 16)] = xj[j, pl.ds(g * 16, 16)] * wq
                return _

            lax.fori_loop(0, C, edge_body, None)

            pltpu.sync_copy(xj, acc.at[dstb], add=True)
            pltpu.sync_copy(exrow, ex_hbm.at[pl.ds(base, C), :])
            return _

        lax.fori_loop(0, NCHUNK, chunk_body, None)
        plsc.subcore_barrier()

        obase = cid * N + sid * ROWS_PT
        pltpu.sync_copy(acc.at[pl.ds(sid * ROWS_PT, ROWS_PT), :],
                        accp_hbm.at[pl.ds(obase, ROWS_PT), :])

        @pl.when(sid == 15)
        def _flush_tail():
            pltpu.sync_copy(acc.at[pl.ds(16 * ROWS_PT, TAIL), :],
                            accp_hbm.at[pl.ds(cid * N + 16 * ROWS_PT, TAIL), :])

        plsc.subcore_barrier()

    return k(xlh, xrh, eatt, src, dst, att_h)


def _sc_den(exs, dst):
    mesh = plsc.VectorSubcoreMesh(core_axis_name="c", subcore_axis_name="s")

    @functools.partial(
        pl.kernel,
        mesh=mesh,
        out_type=jax.ShapeDtypeStruct((2 * H * N, OUT), jnp.float32),
        scratch_types=[
            pltpu.VMEM((C,), jnp.int32),
            pltpu.VMEM((C, 16), jnp.float32),
            pltpu.VMEM((C, OUT), jnp.float32),
            pltpu.VMEM((ZROWS, OUT), jnp.float32),
            pltpu.VMEM_SHARED((N, OUT), jnp.float32),
        ],
    )
    def k(ex0_hbm, ex1_hbm, ex2_hbm, ex3_hbm, dst_hbm, denp_hbm,
          dstb, exc, exw, zb, den):
        cid = lax.axis_index("c")
        sid = lax.axis_index("s")
        wid = cid * 16 + sid
        ebase = wid * EPT
        exhs = (ex0_hbm, ex1_hbm, ex2_hbm, ex3_hbm)

        def zrow(j, _):
            for g in range(OUT // 16):
                zb[j, pl.ds(g * 16, 16)] = jnp.zeros((16,), jnp.float32)
            return _

        lax.fori_loop(0, ZROWS, zrow, None)

        for hh in range(H):
            for t in range(ROWS_PT // ZROWS):
                pltpu.sync_copy(zb, den.at[pl.ds(sid * ROWS_PT + t * ZROWS, ZROWS), :])

            @pl.when(sid == 15)
            def _zero_tail():
                pltpu.sync_copy(zb.at[pl.ds(0, TAIL), :],
                                den.at[pl.ds(16 * ROWS_PT, TAIL), :])

            plsc.subcore_barrier()

            def chunk_body(i, _):
                base = ebase + i * C
                pltpu.sync_copy(dst_hbm.at[pl.ds(base, C)], dstb)
                pltpu.sync_copy(exhs[hh].at[pl.ds(base, C), :], exc)

                def widen(j, _):
                    v = exc[j, pl.ds(0, 16)]
                    for g in range(OUT // 16):
                        exw[j, pl.ds(g * 16, 16)] = v
                    return _

                lax.fori_loop(0, C, widen, None)
                pltpu.sync_copy(exw, den.at[dstb], add=True)
                return _

            lax.fori_loop(0, NCHUNK, chunk_body, None)
            plsc.subcore_barrier()

            obase = (hh * 2 + cid) * N + sid * ROWS_PT
            pltpu.sync_copy(den.at[pl.ds(sid * ROWS_PT, ROWS_PT), :],
                            denp_hbm.at[pl.ds(obase, ROWS_PT), :])

            @pl.when(sid == 15)
            def _flush_tail():
                tbase = (hh * 2 + cid) * N + 16 * ROWS_PT
                pltpu.sync_copy(den.at[pl.ds(16 * ROWS_PT, TAIL), :],
                                denp_hbm.at[pl.ds(tbase, TAIL), :])

            plsc.subcore_barrier()

    return k(*exs, dst)


def _combine_kernel(a0, a1, a2, a3, dp, bias_ref, out_ref):
    out = jnp.zeros(out_ref.shape, jnp.float32)
    for hh, a_ref in enumerate((a0, a1, a2, a3)):
        a = a_ref[0] + a_ref[1]
        d = dp[2 * hh, :, 0] + dp[2 * hh + 1, :, 0]
        out = out + a / (d[:, None] + 1e-16)
    out_ref[...] = out + bias_ref[...]


def _combine(accs, dens, bias):
    BN = 1000
    grid = (N // BN,)
    acc_spec = pl.BlockSpec((2, BN, OUT), lambda i: (0, i, 0))
    den_spec = pl.BlockSpec((2 * H, BN, OUT), lambda i: (0, i, 0))
    return pl.pallas_call(
        _combine_kernel,
        grid=grid,
        in_specs=[acc_spec] * 4 + [den_spec, pl.BlockSpec((1, OUT), lambda i: (0, 0))],
        out_specs=pl.BlockSpec((BN, OUT), lambda i: (i, 0)),
        out_shape=jax.ShapeDtypeStruct((N, OUT), jnp.float32),
    )(*[a.reshape(2, N, OUT) for a in accs],
      dens.reshape(2 * H, N, OUT),
      bias.reshape(1, OUT))


def kernel(x, edge_index, edge_attr, W_l, b_l, W_r, b_r, att, W_e, b_e, bias,
           gate_W, gate_b):
    src = edge_index[0]
    dst = edge_index[1]
    xlh, xrh = _proj(x, W_l, b_l, W_r, b_r)
    eatt = _eatt(edge_attr, W_e, b_e)
    accs, exs = [], []
    for h in range(H):
        a, ex = _sc_head(h, xlh, xrh, eatt, src, dst, att[h])
        accs.append(a)
        exs.append(ex)
    denp = _sc_den(exs, dst)
    return _combine(accs, denp, bias)
